# baseline (speedup 1.0000x reference)
import functools

import jax
import jax.numpy as jnp
from jax import lax
from jax.experimental import pallas as pl
from jax.experimental.pallas import tpu as pltpu
from jax.experimental.pallas import tpu_sc as plsc

VOCAB = 30522
HID = 768
MAX_POS = 512
EPS = 1e-12
B, S = 1024, 512
N = B * S

NUM_WORKERS = 32
TOK_PER_WORKER = N // NUM_WORKERS
W = 32
CHUNKS = TOK_PER_WORKER // W
CHUNKS_PER_SEQ = S // W
NSL = HID // 16


def _allsum(x, iota):
    for k in (1, 2, 4, 8):
        perm = lax.gather(
            x, (iota ^ k)[:, None],
            dimension_numbers=lax.GatherDimensionNumbers(
                offset_dims=(), collapsed_slice_dims=(0,), start_index_map=(0,)),
            slice_sizes=(1,),
            mode=lax.GatherScatterMode.PROMISE_IN_BOUNDS)
        x = x + perm
    return x


def _rsqrt(x):
    i = lax.bitcast_convert_type(x, jnp.int32)
    i = jnp.int32(0x5F3759DF) - (i >> 1)
    y = lax.bitcast_convert_type(i, jnp.float32)
    for _ in range(3):
        y = y * (1.5 - 0.5 * x * y * y)
    return y


def _body(ids_hbm, tt_hbm, word_hbm, pos_hbm, type_hbm, gamma_hbm, beta_hbm,
          out_hbm, idv, ttv, wbuf, pbuf, tbuf, gbuf, bbuf, sem0, sem1):
    nc = 2
    wid = lax.axis_index("s") * nc + lax.axis_index("c")
    tile_base = wid * TOK_PER_WORKER

    pltpu.sync_copy(gamma_hbm, gbuf)
    pltpu.sync_copy(beta_hbm, bbuf)

    def chunk_body(i, _):
        base = tile_base + i * W
        pos_start = (i % CHUNKS_PER_SEQ) * W

        pltpu.sync_copy(ids_hbm.at[pl.ds(base, W)], idv)
        pltpu.sync_copy(tt_hbm.at[pl.ds(base, W)], ttv)
        cw = pltpu.async_copy(word_hbm.at[idv], wbuf, sem0)
        ct = pltpu.async_copy(type_hbm.at[ttv], tbuf, sem1)
        pltpu.sync_copy(pos_hbm.at[pl.ds(pos_start, W)], pbuf)
        cw.wait()
        ct.wait()

        iota = lax.iota(jnp.int32, 16)

        def tok_body(t, _):
            acc = jnp.zeros((16,), jnp.float32)
            acc2 = jnp.zeros((16,), jnp.float32)
            for s in range(NSL):
                sl = pl.ds(s * 16, 16)
                x = wbuf[t, sl] + pbuf[t, sl] + tbuf[t, sl]
                wbuf[t, sl] = x
                acc = acc + x
                acc2 = acc2 + x * x
            tot = _allsum(acc, iota)
            tot2 = _allsum(acc2, iota)
            mean_v = tot * (1.0 / HID)
            var_v = tot2 * (1.0 / HID) - mean_v * mean_v
            inv_v = _rsqrt(var_v + EPS)
            for s in range(NSL):
                sl = pl.ds(s * 16, 16)
                x = wbuf[t, sl]
                wbuf[t, sl] = (x - mean_v) * inv_v * gbuf[sl] + bbuf[sl]
            return _

        lax.fori_loop(0, W, tok_body, None)
        pltpu.sync_copy(wbuf, out_hbm.at[pl.ds(base, W)])
        return _

    lax.fori_loop(0, CHUNKS, chunk_body, None)


@jax.jit
def _run(ids_flat, tt_flat, word_emb, pos_emb, type_emb, gamma, beta):
    mesh = plsc.VectorSubcoreMesh(core_axis_name="c", subcore_axis_name="s")
    f = pl.kernel(
        _body,
        out_type=jax.ShapeDtypeStruct((N, HID), jnp.float32),
        mesh=mesh,
        scratch_types=[
            pltpu.VMEM((W,), jnp.int32),
            pltpu.VMEM((W,), jnp.int32),
            pltpu.VMEM((W, HID), jnp.float32),
            pltpu.VMEM((W, HID), jnp.float32),
            pltpu.VMEM((W, HID), jnp.float32),
            pltpu.VMEM((HID,), jnp.float32),
            pltpu.VMEM((HID,), jnp.float32),
            pltpu.SemaphoreType.DMA,
            pltpu.SemaphoreType.DMA,
        ],
    )
    return f(ids_flat, tt_flat, word_emb, pos_emb, type_emb, gamma, beta)


def kernel(input_ids, token_type_ids, word_emb, pos_emb, type_emb, gamma, beta):
    ids_flat = input_ids.reshape(N).astype(jnp.int32)
    tt_flat = token_type_ids.reshape(N).astype(jnp.int32)
    out = _run(ids_flat, tt_flat, word_emb, pos_emb, type_emb, gamma, beta)
    return out.reshape(B, S, HID)


# speedup vs baseline: 2.3872x; 2.3872x over previous
import jax
import jax.numpy as jnp
from jax import lax
from jax.experimental import pallas as pl
from jax.experimental.pallas import tpu as pltpu
from jax.experimental.pallas import tpu_sc as plsc

VOCAB = 30522
HID = 768
MAX_POS = 512
EPS = 1e-12
B, S = 1024, 512
N = B * S

NUM_WORKERS = 32
TPW = N // NUM_WORKERS
W = 16
CH = TPW // W
NSL = HID // 16


def _allsum(x, iota):
    for k in (1, 2, 4, 8):
        perm = lax.gather(
            x, (iota ^ k)[:, None],
            dimension_numbers=lax.GatherDimensionNumbers(
                offset_dims=(), collapsed_slice_dims=(0,), start_index_map=(0,)),
            slice_sizes=(1,),
            mode=lax.GatherScatterMode.PROMISE_IN_BOUNDS)
        x = x + perm
    return x


def _rsqrt(x):
    i = lax.bitcast_convert_type(x, jnp.int32)
    i = jnp.int32(0x5F3759DF) - (i >> 1)
    y = lax.bitcast_convert_type(i, jnp.float32)
    for _ in range(3):
        y = y * (1.5 - 0.5 * x * y * y)
    return y


def _body(ids_hbm, cidx_hbm, word_hbm, comb_hbm, gamma_hbm, beta_hbm, out_hbm,
          idsv, cidxv, w0, w1, c0, c1, o0, o1, gbuf, bbuf,
          sw0, sw1, sc0, sc1, so0, so1):
    nc = 2
    wid = lax.axis_index("s") * nc + lax.axis_index("c")
    base = wid * TPW

    pltpu.sync_copy(gamma_hbm, gbuf)
    pltpu.sync_copy(beta_hbm, bbuf)
    pltpu.sync_copy(ids_hbm.at[pl.ds(base, TPW)], idsv)
    pltpu.sync_copy(cidx_hbm.at[pl.ds(base, TPW)], cidxv)

    wb, cb, ob = (w0, w1), (c0, c1), (o0, o1)
    sW, sC, sO = (sw0, sw1), (sc0, sc1), (so0, so1)
    iota = lax.iota(jnp.int32, 16)

    def start_gather(i, p):
        pltpu.async_copy(word_hbm.at[idsv.at[pl.ds(i * W, W)]], wb[p], sW[p])
        pltpu.async_copy(comb_hbm.at[cidxv.at[pl.ds(i * W, W)]], cb[p], sC[p])

    def wait_gather(i, p):
        pltpu.make_async_copy(
            word_hbm.at[idsv.at[pl.ds(i * W, W)]], wb[p], sW[p]).wait()
        pltpu.make_async_copy(
            comb_hbm.at[cidxv.at[pl.ds(i * W, W)]], cb[p], sC[p]).wait()

    def start_writeback(i, p):
        pltpu.async_copy(ob[p], out_hbm.at[pl.ds(base + i * W, W)], sO[p])

    def wait_writeback(i, p):
        pltpu.make_async_copy(
            ob[p], out_hbm.at[pl.ds(base + i * W, W)], sO[p]).wait()

    def compute(p):
        wbp, cbp, obp = wb[p], cb[p], ob[p]

        def tok(t, _):
            acc = jnp.zeros((16,), jnp.float32)
            acc2 = jnp.zeros((16,), jnp.float32)
            xs = []
            for s in range(NSL):
                sl = pl.ds(s * 16, 16)
                x = wbp[t, sl] + cbp[t, sl]
                xs.append(x)
                acc = acc + x
                acc2 = acc2 + x * x
            tot = _allsum(acc, iota)
            tot2 = _allsum(acc2, iota)
            mean = tot * (1.0 / HID)
            var = tot2 * (1.0 / HID) - mean * mean
            inv = _rsqrt(var + EPS)
            for s in range(NSL):
                sl = pl.ds(s * 16, 16)
                obp[t, sl] = (xs[s] - mean) * inv * gbuf[sl] + bbuf[sl]
            return _

        lax.fori_loop(0, W, tok, None)

    start_gather(0, 0)
    start_gather(1, 1)
    for p in (0, 1):
        i = p
        wait_gather(i, p)
        compute(p)
        start_writeback(i, p)
        start_gather(i + 2, p)

    def pair(j, _):
        for p in (0, 1):
            i = 2 * j + p
            wait_gather(i, p)
            wait_writeback(i - 2, p)
            compute(p)
            start_writeback(i, p)
            start_gather(i + 2, p)
        return _

    lax.fori_loop(1, CH // 2 - 1, pair, None)

    for p in (0, 1):
        i = CH - 2 + p
        wait_gather(i, p)
        wait_writeback(i - 2, p)
        compute(p)
        start_writeback(i, p)
    for p in (0, 1):
        wait_writeback(CH - 2 + p, p)


@jax.jit
def _run(ids_flat, cidx_flat, word_emb, comb, gamma, beta):
    mesh = plsc.VectorSubcoreMesh(core_axis_name="c", subcore_axis_name="s")
    f = pl.kernel(
        _body,
        out_type=jax.ShapeDtypeStruct((N, HID), jnp.float32),
        mesh=mesh,
        scratch_types=[
            pltpu.VMEM((TPW,), jnp.int32),
            pltpu.VMEM((TPW,), jnp.int32),
            pltpu.VMEM((W, HID), jnp.float32),
            pltpu.VMEM((W, HID), jnp.float32),
            pltpu.VMEM((W, HID), jnp.float32),
            pltpu.VMEM((W, HID), jnp.float32),
            pltpu.VMEM((W, HID), jnp.float32),
            pltpu.VMEM((W, HID), jnp.float32),
            pltpu.VMEM((HID,), jnp.float32),
            pltpu.VMEM((HID,), jnp.float32),
            pltpu.SemaphoreType.DMA,
            pltpu.SemaphoreType.DMA,
            pltpu.SemaphoreType.DMA,
            pltpu.SemaphoreType.DMA,
            pltpu.SemaphoreType.DMA,
            pltpu.SemaphoreType.DMA,
        ],
    )
    return f(ids_flat, cidx_flat, word_emb, comb, gamma, beta)


def kernel(input_ids, token_type_ids, word_emb, pos_emb, type_emb, gamma, beta):
    ids_flat = input_ids.reshape(N).astype(jnp.int32)
    tt = token_type_ids.astype(jnp.int32)
    comb = (pos_emb[:, None, :] + type_emb[None, :, :]).reshape(2 * S, HID)
    cidx_flat = (2 * jnp.arange(S, dtype=jnp.int32)[None, :] + tt).reshape(N)
    out = _run(ids_flat, cidx_flat, word_emb, comb, gamma, beta)
    return out.reshape(B, S, HID)
